# baseline (speedup 1.0000x reference)
import functools

import jax
import jax.numpy as jnp
from jax import lax
from jax.experimental import pallas as pl
from jax.experimental.pallas import tpu as pltpu
from jax.experimental.pallas import tpu_sc as plsc

N_NODES = 10000
N_EDGES = 320000
D = 128
DH = D // 2
LANES = 16

NC = 2
NS = 16
CHUNK = 128
NB = 3
NCH = -(-N_EDGES // (NS * CHUNK * NB)) * NB
EPW = NCH * CHUNK
EPAD = NS * EPW

N_PAD = 10240
ROWS_PER_TILE = N_PAD // NS

_GATHER_DN = lax.GatherDimensionNumbers(
    offset_dims=(), collapsed_slice_dims=(0,), start_index_map=(0,))


def _sc_segment_sum(yh, src, dst, w):
  mesh = plsc.VectorSubcoreMesh(core_axis_name="c", subcore_axis_name="s")

  @functools.partial(
      pl.kernel,
      out_type=jax.ShapeDtypeStruct((NC, N_PAD, DH), jnp.float32),
      mesh=mesh,
      scratch_types=[
          pltpu.VMEM_SHARED((N_PAD, DH), jnp.float32),
          pltpu.VMEM((NCH, CHUNK), jnp.int32),
          pltpu.VMEM((NCH, CHUNK), jnp.int32),
          pltpu.VMEM((NCH, CHUNK), jnp.float32),
      ] + [pltpu.VMEM((CHUNK, DH), jnp.float32)] * NB
        + [pltpu.SemaphoreType.DMA] * (2 * NB),
      compiler_params=pltpu.CompilerParams(use_tc_tiling_on_sc=False),
  )
  def k(y_hbm, src_hbm, dst_hbm, w_hbm, out_hbm, acc, src_all, dst_all, w_all,
        *bufs_and_sems):
    rows = bufs_and_sems[:NB]
    gsem = bufs_and_sems[NB:2 * NB]
    ssem = bufs_and_sems[2 * NB:3 * NB]
    c = lax.axis_index("c")
    s = lax.axis_index("s")

    pltpu.sync_copy(src_hbm.at[s], src_all)
    pltpu.sync_copy(dst_hbm.at[s], dst_all)
    pltpu.sync_copy(w_hbm.at[s], w_all)

    def zero_body(rr, carry):
      for j in range(DH // LANES):
        rows[0][rr, pl.ds(j * LANES, LANES)] = jnp.zeros((LANES,), jnp.float32)
      return carry
    lax.fori_loop(0, CHUNK, zero_body, 0)
    base = s * ROWS_PER_TILE
    for t in range(ROWS_PER_TILE // CHUNK):
      pltpu.sync_copy(rows[0], acc.at[pl.ds(base + t * CHUNK, CHUNK)])

    for b in range(NB):
      pltpu.async_copy(y_hbm.at[c].at[src_all.at[b]], rows[b], gsem[b])

    plsc.subcore_barrier()

    def scale_chunk(cur, rbuf):
      def scale_group(g, carry):
        w16 = w_all[cur, pl.ds(g * LANES, LANES)]
        for i in range(LANES):
          lane = jnp.broadcast_to(jnp.int32(i), (LANES,))
          wb = lax.gather(w16, lane[:, None], _GATHER_DN, slice_sizes=(1,),
                          mode=lax.GatherScatterMode.PROMISE_IN_BOUNDS)
          e = g * LANES + i
          for j in range(DH // LANES):
            sl = pl.ds(j * LANES, LANES)
            rbuf[e, sl] = rbuf[e, sl] * wb
        return carry
      lax.fori_loop(0, CHUNK // LANES, scale_group, 0)

    def outer(o, carry):
      ch = o * NB
      for b in range(NB):
        cur = ch + b
        pltpu.make_async_copy(y_hbm.at[c].at[src_all.at[cur]], rows[b],
                              gsem[b]).wait()
        scale_chunk(cur, rows[b])
        pb = (b - 1) % NB
        prev = cur - 1
        nxt = prev + NB
        cond = nxt < NCH if b > 0 else jnp.logical_and(prev >= 0, nxt < NCH)
        @pl.when(cond)
        def _():
          pltpu.make_async_copy(rows[pb], acc.at[dst_all.at[prev]],
                                ssem[pb]).wait()
          pltpu.async_copy(y_hbm.at[c].at[src_all.at[nxt]], rows[pb], gsem[pb])
        pltpu.async_copy(rows[b], acc.at[dst_all.at[cur]], ssem[b], add=True)
      return carry
    lax.fori_loop(0, NCH // NB, outer, 0)

    for b in range(NB):
      last = NCH - NB + b
      pltpu.make_async_copy(rows[b], acc.at[dst_all.at[last]], ssem[b]).wait()

    plsc.subcore_barrier()
    pltpu.sync_copy(acc.at[pl.ds(base, ROWS_PER_TILE)],
                    out_hbm.at[c, pl.ds(base, ROWS_PER_TILE)])

  return k(yh, src, dst, w)


_BR = 400
_DOT_DN = (((1,), (1,)), ((), ()))


def _tc_prep(x, W_rel, W_root, b_rel):
  def body(x_ref, wrel_ref, wroot_ref, b_ref, y_ref, r_ref):
    xb = x_ref[...]
    y = lax.dot_general(xb, wrel_ref[...], _DOT_DN,
                        preferred_element_type=jnp.float32)
    y_ref[0] = y[:, :DH]
    y_ref[1] = y[:, DH:]
    r_ref[...] = lax.dot_general(xb, wroot_ref[...], _DOT_DN,
                                 preferred_element_type=jnp.float32) + b_ref[...]

  return pl.pallas_call(
      body,
      grid=(N_NODES // _BR,),
      in_specs=[
          pl.BlockSpec((_BR, D), lambda i: (i, 0)),
          pl.BlockSpec((D, D), lambda i: (0, 0)),
          pl.BlockSpec((D, D), lambda i: (0, 0)),
          pl.BlockSpec((1, D), lambda i: (0, 0)),
      ],
      out_specs=[
          pl.BlockSpec((NC, _BR, DH), lambda i: (0, i, 0)),
          pl.BlockSpec((_BR, D), lambda i: (i, 0)),
      ],
      out_shape=[jax.ShapeDtypeStruct((NC, N_NODES, DH), jnp.float32),
                 jax.ShapeDtypeStruct((N_NODES, D), jnp.float32)],
  )(x, W_rel, W_root, b_rel.reshape(1, D))


def _tc_combine_prep(p, r1, W_rel, W_root, b_rel):
  def body(p_ref, r1_ref, wrel_ref, wroot_ref, b_ref, y_ref, r_ref):
    agg = jnp.concatenate([p_ref[0], p_ref[1]], axis=1)
    h = jnp.maximum(agg + r1_ref[...], 0.0)
    y = lax.dot_general(h, wrel_ref[...], _DOT_DN,
                        preferred_element_type=jnp.float32)
    y_ref[0] = y[:, :DH]
    y_ref[1] = y[:, DH:]
    r_ref[...] = lax.dot_general(h, wroot_ref[...], _DOT_DN,
                                 preferred_element_type=jnp.float32) + b_ref[...]

  return pl.pallas_call(
      body,
      grid=(N_NODES // _BR,),
      in_specs=[
          pl.BlockSpec((NC, _BR, DH), lambda i: (0, i, 0)),
          pl.BlockSpec((_BR, D), lambda i: (i, 0)),
          pl.BlockSpec((D, D), lambda i: (0, 0)),
          pl.BlockSpec((D, D), lambda i: (0, 0)),
          pl.BlockSpec((1, D), lambda i: (0, 0)),
      ],
      out_specs=[
          pl.BlockSpec((NC, _BR, DH), lambda i: (0, i, 0)),
          pl.BlockSpec((_BR, D), lambda i: (i, 0)),
      ],
      out_shape=[jax.ShapeDtypeStruct((NC, N_NODES, DH), jnp.float32),
                 jax.ShapeDtypeStruct((N_NODES, D), jnp.float32)],
  )(p, r1, W_rel, W_root, b_rel.reshape(1, D))


def _tc_final(q, r2):
  def body(q_ref, r2_ref, out_ref):
    out_ref[...] = jnp.concatenate([q_ref[0], q_ref[1]], axis=1) + r2_ref[...]

  return pl.pallas_call(
      body,
      grid=(N_NODES // _BR,),
      in_specs=[
          pl.BlockSpec((NC, _BR, DH), lambda i: (0, i, 0)),
          pl.BlockSpec((_BR, D), lambda i: (i, 0)),
      ],
      out_specs=pl.BlockSpec((_BR, D), lambda i: (i, 0)),
      out_shape=jax.ShapeDtypeStruct((N_NODES, D), jnp.float32),
  )(q, r2)


def kernel(x, edge_index, edge_weight, W1_rel, b1_rel, W1_root,
           W2_rel, b2_rel, W2_root):
  pad = EPAD - N_EDGES
  src = jnp.pad(edge_index[0].astype(jnp.int32), (0, pad)).reshape(
      NS, NCH, CHUNK)
  dst = jnp.pad(edge_index[1].astype(jnp.int32), (0, pad)).reshape(
      NS, NCH, CHUNK)
  w = jnp.pad(edge_weight.astype(jnp.float32), (0, pad)).reshape(
      NS, NCH, CHUNK)

  y1, r1 = _tc_prep(x, W1_rel, W1_root, b1_rel)
  p1 = _sc_segment_sum(y1, src, dst, w)[:, :N_NODES]
  y2, r2 = _tc_combine_prep(p1, r1, W2_rel, W2_root, b2_rel)
  p2 = _sc_segment_sum(y2, src, dst, w)[:, :N_NODES]
  return _tc_final(p2, r2)


# speedup vs baseline: 1.7075x; 1.7075x over previous
import functools

import jax
import jax.numpy as jnp
from jax import lax
from jax.experimental import pallas as pl
from jax.experimental.pallas import tpu as pltpu
from jax.experimental.pallas import tpu_sc as plsc

N_NODES = 10000
N_EDGES = 320000
D = 128
LANES = 16

NC = 2
NS = 16
NW = NC * NS
CHUNK = 80
NB = 3
NCH = -(-N_EDGES // (NW * CHUNK * NB)) * NB
EPW = NCH * CHUNK
EPAD = NW * EPW

N_PAD = 10112
ROWS_PER_TILE = N_PAD // NS

_GATHER_DN = lax.GatherDimensionNumbers(
    offset_dims=(), collapsed_slice_dims=(0,), start_index_map=(0,))


def _sc_segment_sum(y, src, dst, w):
  mesh = plsc.VectorSubcoreMesh(core_axis_name="c", subcore_axis_name="s")

  @functools.partial(
      pl.kernel,
      out_type=jax.ShapeDtypeStruct((NC, N_PAD, D), jnp.float32),
      mesh=mesh,
      scratch_types=[
          pltpu.VMEM_SHARED((N_PAD, D), jnp.float32),
          pltpu.VMEM((NCH, CHUNK), jnp.int32),
      ] + [pltpu.VMEM((CHUNK, D), jnp.float32)] * NB
        + [pltpu.VMEM((CHUNK,), jnp.int32)] * NB
        + [pltpu.VMEM((CHUNK,), jnp.float32)] * NB
        + [pltpu.SemaphoreType.DMA] * (3 * NB),
  )
  def k(y_hbm, src_hbm, dst_hbm, w_hbm, out_hbm, acc, src_all, *rest):
    rows = rest[:NB]
    dstb = rest[NB:2 * NB]
    wbuf = rest[2 * NB:3 * NB]
    gsem = rest[3 * NB:4 * NB]
    ssem = rest[4 * NB:5 * NB]
    msem = rest[5 * NB:6 * NB]
    c = lax.axis_index("c")
    s = lax.axis_index("s")

    pltpu.sync_copy(src_hbm.at[c, s], src_all)

    def zero_body(rr, carry):
      for j in range(D // LANES):
        rows[0][rr, pl.ds(j * LANES, LANES)] = jnp.zeros((LANES,), jnp.float32)
      return carry
    lax.fori_loop(0, CHUNK, zero_body, 0)
    base = s * ROWS_PER_TILE
    nfull = ROWS_PER_TILE // CHUNK
    for t in range(nfull):
      pltpu.sync_copy(rows[0], acc.at[pl.ds(base + t * CHUNK, CHUNK)])
    tail = ROWS_PER_TILE - nfull * CHUNK
    pltpu.sync_copy(rows[0].at[pl.ds(0, tail)],
                    acc.at[pl.ds(base + nfull * CHUNK, tail)])

    for b in range(NB):
      pltpu.async_copy(dst_hbm.at[c, s, b], dstb[b], msem[b])
      pltpu.async_copy(w_hbm.at[c, s, b], wbuf[b], msem[b])
      pltpu.async_copy(y_hbm.at[src_all.at[b]], rows[b], gsem[b])

    plsc.subcore_barrier()

    def scale_chunk(b):
      def scale_group(g, carry):
        w16 = wbuf[b][pl.ds(g * LANES, LANES)]
        for i in range(LANES):
          lane = jnp.broadcast_to(jnp.int32(i), (LANES,))
          wb = lax.gather(w16, lane[:, None], _GATHER_DN, slice_sizes=(1,),
                          mode=lax.GatherScatterMode.PROMISE_IN_BOUNDS)
          e = g * LANES + i
          for j in range(D // LANES):
            sl = pl.ds(j * LANES, LANES)
            rows[b][e, sl] = rows[b][e, sl] * wb
        return carry
      lax.fori_loop(0, CHUNK // LANES, scale_group, 0)

    def outer(o, carry):
      ch = o * NB
      for b in range(NB):
        cur = ch + b
        pltpu.make_async_copy(y_hbm.at[src_all.at[cur]], rows[b],
                              gsem[b]).wait()
        pltpu.make_async_copy(dst_hbm.at[c, s, cur], dstb[b], msem[b]).wait()
        pltpu.make_async_copy(w_hbm.at[c, s, cur], wbuf[b], msem[b]).wait()
        scale_chunk(b)
        pb = (b - 1) % NB
        prev = cur - 1
        nxt = prev + NB
        cond = nxt < NCH if b > 0 else jnp.logical_and(prev >= 0, nxt < NCH)
        @pl.when(cond)
        def _():
          pltpu.make_async_copy(rows[pb], acc.at[dstb[pb]], ssem[pb]).wait()
          pltpu.async_copy(dst_hbm.at[c, s, nxt], dstb[pb], msem[pb])
          pltpu.async_copy(w_hbm.at[c, s, nxt], wbuf[pb], msem[pb])
          pltpu.async_copy(y_hbm.at[src_all.at[nxt]], rows[pb], gsem[pb])
        pltpu.async_copy(rows[b], acc.at[dstb[b]], ssem[b], add=True)
      return carry
    lax.fori_loop(0, NCH // NB, outer, 0)

    for b in range(NB):
      pltpu.make_async_copy(rows[b], acc.at[dstb[b]], ssem[b]).wait()

    plsc.subcore_barrier()
    pltpu.sync_copy(acc.at[pl.ds(base, ROWS_PER_TILE)],
                    out_hbm.at[c, pl.ds(base, ROWS_PER_TILE)])

  return k(y, src, dst, w)


_BR = 400
_DOT_DN = (((1,), (1,)), ((), ()))


def _tc_prep(x, W_rel, W_root, b_rel):
  def body(x_ref, wrel_ref, wroot_ref, b_ref, y_ref, r_ref):
    xb = x_ref[...]
    y_ref[...] = lax.dot_general(xb, wrel_ref[...], _DOT_DN,
                                 preferred_element_type=jnp.float32)
    r_ref[...] = lax.dot_general(xb, wroot_ref[...], _DOT_DN,
                                 preferred_element_type=jnp.float32) + b_ref[...]

  return pl.pallas_call(
      body,
      grid=(N_NODES // _BR,),
      in_specs=[
          pl.BlockSpec((_BR, D), lambda i: (i, 0)),
          pl.BlockSpec((D, D), lambda i: (0, 0)),
          pl.BlockSpec((D, D), lambda i: (0, 0)),
          pl.BlockSpec((1, D), lambda i: (0, 0)),
      ],
      out_specs=[
          pl.BlockSpec((_BR, D), lambda i: (i, 0)),
          pl.BlockSpec((_BR, D), lambda i: (i, 0)),
      ],
      out_shape=[jax.ShapeDtypeStruct((N_NODES, D), jnp.float32)] * 2,
  )(x, W_rel, W_root, b_rel.reshape(1, D))


def _tc_combine_prep(p, r1, W_rel, W_root, b_rel):
  def body(p_ref, r1_ref, wrel_ref, wroot_ref, b_ref, y_ref, r_ref):
    h = jnp.maximum(p_ref[0] + p_ref[1] + r1_ref[...], 0.0)
    y_ref[...] = lax.dot_general(h, wrel_ref[...], _DOT_DN,
                                 preferred_element_type=jnp.float32)
    r_ref[...] = lax.dot_general(h, wroot_ref[...], _DOT_DN,
                                 preferred_element_type=jnp.float32) + b_ref[...]

  return pl.pallas_call(
      body,
      grid=(N_NODES // _BR,),
      in_specs=[
          pl.BlockSpec((NC, _BR, D), lambda i: (0, i, 0)),
          pl.BlockSpec((_BR, D), lambda i: (i, 0)),
          pl.BlockSpec((D, D), lambda i: (0, 0)),
          pl.BlockSpec((D, D), lambda i: (0, 0)),
          pl.BlockSpec((1, D), lambda i: (0, 0)),
      ],
      out_specs=[
          pl.BlockSpec((_BR, D), lambda i: (i, 0)),
          pl.BlockSpec((_BR, D), lambda i: (i, 0)),
      ],
      out_shape=[jax.ShapeDtypeStruct((N_NODES, D), jnp.float32)] * 2,
  )(p, r1, W_rel, W_root, b_rel.reshape(1, D))


def _tc_final(q, r2):
  def body(q_ref, r2_ref, out_ref):
    out_ref[...] = q_ref[0] + q_ref[1] + r2_ref[...]

  return pl.pallas_call(
      body,
      grid=(N_NODES // _BR,),
      in_specs=[
          pl.BlockSpec((NC, _BR, D), lambda i: (0, i, 0)),
          pl.BlockSpec((_BR, D), lambda i: (i, 0)),
      ],
      out_specs=pl.BlockSpec((_BR, D), lambda i: (i, 0)),
      out_shape=jax.ShapeDtypeStruct((N_NODES, D), jnp.float32),
  )(q, r2)


def kernel(x, edge_index, edge_weight, W1_rel, b1_rel, W1_root,
           W2_rel, b2_rel, W2_root):
  pad = EPAD - N_EDGES
  src = jnp.pad(edge_index[0].astype(jnp.int32), (0, pad)).reshape(
      NC, NS, NCH, CHUNK)
  dst = jnp.pad(edge_index[1].astype(jnp.int32), (0, pad)).reshape(
      NC, NS, NCH, CHUNK)
  w = jnp.pad(edge_weight.astype(jnp.float32), (0, pad)).reshape(
      NC, NS, NCH, CHUNK)

  y1, r1 = _tc_prep(x, W1_rel, W1_root, b1_rel)
  p1 = _sc_segment_sum(y1, src, dst, w)[:, :N_NODES]
  y2, r2 = _tc_combine_prep(p1, r1, W2_rel, W2_root, b2_rel)
  p2 = _sc_segment_sum(y2, src, dst, w)[:, :N_NODES]
  return _tc_final(p2, r2)
